# baseline (speedup 1.0000x reference)
import functools

import jax
import jax.numpy as jnp
from jax import lax
from jax.experimental import pallas as pl
from jax.experimental.pallas import tpu as pltpu
from jax.experimental.pallas import tpu_sc as plsc


_TILES = 16


@functools.lru_cache(maxsize=None)
def _make_seg_sum(n_nodes: int, n_edges: int, width: int, edge_split: bool):
    n_workers = 2 * _TILES if edge_split else _TILES
    chunk = 40 if edge_split else 80
    phases = 5
    ep = n_edges // n_workers
    nch = ep // chunk
    pnch = nch // phases
    assert ep % chunk == 0 and nch % phases == 0 and pnch % 2 == 0
    rpt = n_nodes // _TILES
    assert rpt * _TILES == n_nodes
    vecs_per_row = chunk // 16

    mesh = plsc.VectorSubcoreMesh(core_axis_name="c", subcore_axis_name="s")

    @functools.partial(
        pl.kernel,
        out_type=[
            jax.ShapeDtypeStruct((_TILES, rpt, width), jnp.float32),
            jax.ShapeDtypeStruct((_TILES, rpt, width), jnp.float32),
        ],
        mesh=mesh,
        scratch_types=[
            pltpu.VMEM((pnch, chunk), jnp.int32),
            pltpu.VMEM((pnch, chunk), jnp.int32),
            pltpu.VMEM((chunk, width), jnp.float32),
            pltpu.VMEM((chunk, width), jnp.float32),
            pltpu.VMEM_SHARED((n_nodes, width), jnp.float32),
            pltpu.SemaphoreType.DMA,
            pltpu.SemaphoreType.DMA,
            pltpu.SemaphoreType.DMA,
            pltpu.SemaphoreType.DMA,
        ],
    )
    def seg_kernel(hview, src_hbm, dst_hbm, zero_hbm, out_lo, out_hi,
                   src_v, dst_v, gb0, gb1, accum, sem0, sem1, ssem0, ssem1):
        c = lax.axis_index("c")
        s = lax.axis_index("s")
        w = c * _TILES + s if edge_split else s

        pltpu.sync_copy(zero_hbm, accum.at[pl.ds(s * rpt, rpt)])
        plsc.subcore_barrier()

        def _phase(p, _):
            pltpu.sync_copy(src_hbm.at[w, p], src_v)
            pltpu.sync_copy(dst_hbm.at[w, p], dst_v)

            if not edge_split:
                def _xform_row(j, _):
                    def _xform_vec(l, _):
                        v = src_v[j, pl.ds(l * 16, 16)]
                        src_v[j, pl.ds(l * 16, 16)] = v + v + c
                        return 0
                    return lax.fori_loop(0, vecs_per_row, _xform_vec, 0)
                lax.fori_loop(0, pnch, _xform_row, 0)

            pltpu.async_copy(hview.at[src_v.at[0]], gb0, sem0)
            pltpu.async_copy(hview.at[src_v.at[1]], gb1, sem1)

            def _body(kk, _):
                c0 = 2 * kk
                c1 = c0 + 1
                pltpu.make_async_copy(hview.at[src_v.at[c0]], gb0,
                                      sem0).wait()
                s0 = pltpu.async_copy(gb0, accum.at[dst_v.at[c0]], ssem0,
                                      add=True)
                pltpu.make_async_copy(hview.at[src_v.at[c1]], gb1,
                                      sem1).wait()
                s1 = pltpu.async_copy(gb1, accum.at[dst_v.at[c1]], ssem1,
                                      add=True)
                s0.wait()
                nxt0 = jnp.minimum(c0 + 2, pnch - 1)
                pltpu.async_copy(hview.at[src_v.at[nxt0]], gb0, sem0)
                s1.wait()
                nxt1 = jnp.minimum(c1 + 2, pnch - 1)
                pltpu.async_copy(hview.at[src_v.at[nxt1]], gb1, sem1)
                return 0

            lax.fori_loop(0, pnch // 2, _body, 0)
            pltpu.make_async_copy(hview.at[src_v.at[0]], gb0, sem0).wait()
            pltpu.make_async_copy(hview.at[src_v.at[0]], gb1, sem1).wait()
            return 0

        lax.fori_loop(0, phases, _phase, 0)
        plsc.subcore_barrier()

        @pl.when(c == 0)
        def _():
            pltpu.sync_copy(accum.at[pl.ds(s * rpt, rpt)], out_lo.at[s])

        @pl.when(c == 1)
        def _():
            pltpu.sync_copy(accum.at[pl.ds(s * rpt, rpt)], out_hi.at[s])

    return seg_kernel


_ROWS = 1000


def _gin_mlp_body(h_ref, a0_ref, a1_ref, w1_ref, b1_ref, w2_ref, b2_ref,
                  o_ref, *, concat):
    if concat:
        a = h_ref[...] + jnp.concatenate([a0_ref[...], a1_ref[...]], axis=1)
    else:
        a = h_ref[...] + a0_ref[...] + a1_ref[...]
    z = jnp.dot(a, w1_ref[...], preferred_element_type=jnp.float32)
    z = jnp.maximum(z + b1_ref[...], 0.0)
    o = jnp.dot(z, w2_ref[...], preferred_element_type=jnp.float32)
    o_ref[...] = jnp.maximum(o + b2_ref[...], 0.0)


def _gin_mlp(h, a0, a1, w1f, b1f, w2, b2, concat):
    n, d = h.shape
    hh = w2.shape[0]
    aw = a0.shape[1]
    grid = (n // _ROWS,)
    return pl.pallas_call(
        functools.partial(_gin_mlp_body, concat=concat),
        grid=grid,
        in_specs=[
            pl.BlockSpec((_ROWS, d), lambda i: (i, 0)),
            pl.BlockSpec((_ROWS, aw), lambda i: (i, 0)),
            pl.BlockSpec((_ROWS, aw), lambda i: (i, 0)),
            pl.BlockSpec((d, hh), lambda i: (0, 0)),
            pl.BlockSpec((1, hh), lambda i: (0, 0)),
            pl.BlockSpec((hh, hh), lambda i: (0, 0)),
            pl.BlockSpec((1, hh), lambda i: (0, 0)),
        ],
        out_specs=pl.BlockSpec((_ROWS, hh), lambda i: (i, 0)),
        out_shape=jax.ShapeDtypeStruct((n, hh), jnp.float32),
    )(h, a0, a1, w1f, b1f.reshape(1, -1), w2, b2.reshape(1, -1))


def _pool_mlp_body(h_ref, b_ref, w1_ref, b1_ref, w2_ref, b2_ref,
                   pooled_ref, out_ref):
    i = pl.program_id(0)
    nblk = pl.num_programs(0)
    g = pooled_ref.shape[0]
    seg = b_ref[0, 0, :]
    iota = lax.broadcasted_iota(jnp.int32, (g, seg.shape[0]), 0)
    onehot = (iota == seg[None, :]).astype(jnp.float32)
    part = jnp.dot(onehot, h_ref[...], preferred_element_type=jnp.float32)

    @pl.when(i == 0)
    def _():
        pooled_ref[...] = jnp.zeros_like(pooled_ref)

    pooled_ref[...] += part

    @pl.when(i == nblk - 1)
    def _():
        p = pooled_ref[...]
        z = jnp.dot(p, w1_ref[...], preferred_element_type=jnp.float32)
        z = jnp.maximum(z + b1_ref[...], 0.0)
        o = jnp.dot(z, w2_ref[...], preferred_element_type=jnp.float32)
        out_ref[...] = o + b2_ref[...]


def _pool_mlp(h, batch, n_graphs, w1, b1, w2, b2):
    n, d = h.shape
    out_d = w2.shape[1]
    grid = (n // _ROWS,)
    batch3 = batch.reshape(n // _ROWS, 1, _ROWS)
    pooled, out = pl.pallas_call(
        _pool_mlp_body,
        grid=grid,
        in_specs=[
            pl.BlockSpec((_ROWS, d), lambda i: (i, 0)),
            pl.BlockSpec((1, 1, _ROWS), lambda i: (i, 0, 0)),
            pl.BlockSpec((d, d), lambda i: (0, 0)),
            pl.BlockSpec((1, d), lambda i: (0, 0)),
            pl.BlockSpec((d, out_d), lambda i: (0, 0)),
            pl.BlockSpec((1, out_d), lambda i: (0, 0)),
        ],
        out_specs=[
            pl.BlockSpec((n_graphs, d), lambda i: (0, 0)),
            pl.BlockSpec((n_graphs, out_d), lambda i: (0, 0)),
        ],
        out_shape=[
            jax.ShapeDtypeStruct((n_graphs, d), jnp.float32),
            jax.ShapeDtypeStruct((n_graphs, out_d), jnp.float32),
        ],
    )(h, batch3, w1, b1.reshape(1, -1), w2, b2.reshape(1, -1))
    return out, pooled


def kernel(x, edge_index, batch,
           l0_W1, l0_b1, l0_g, l0_bb, l0_m, l0_v, l0_W2, l0_b2,
           l1_W1, l1_b1, l1_g, l1_bb, l1_m, l1_v, l1_W2, l1_b2,
           l2_W1, l2_b1, l2_g, l2_bb, l2_m, l2_v, l2_W2, l2_b2,
           mlp_W1, mlp_b1, mlp_W2, mlp_b2):
    n, _ = x.shape
    e = edge_index.shape[1]
    n_graphs = 64

    src, dst = edge_index[0], edge_index[1]

    layers = [
        (l0_W1, l0_b1, l0_g, l0_bb, l0_m, l0_v, l0_W2, l0_b2),
        (l1_W1, l1_b1, l1_g, l1_bb, l1_m, l1_v, l1_W2, l1_b2),
        (l2_W1, l2_b1, l2_g, l2_bb, l2_m, l2_v, l2_W2, l2_b2),
    ]

    h = x
    for (W1, b1, g, bb, m, v, W2, b2) in layers:
        d = h.shape[1]
        half = d // 2
        s = g / jnp.sqrt(v + 1e-5)
        W1f = W1 * s[None, :]
        b1f = (b1 - m) * s + bb
        edge_split = (half % 128 != 0)
        if edge_split:
            width, n_workers, chunk = d, 2 * _TILES, 40
            hsrc = h
        else:
            width, n_workers, chunk = half, _TILES, 80
            hsrc = h.reshape(2 * n, half)
        phases = 5
        pnch = e // n_workers // chunk // phases
        src3d = src.reshape(n_workers, phases, pnch, chunk)
        dst3d = dst.reshape(n_workers, phases, pnch, chunk)
        zero = jnp.zeros((n // _TILES, width), jnp.float32)
        seg = _make_seg_sum(n, e, width, edge_split)
        a0, a1 = seg(hsrc, src3d, dst3d, zero)
        h = _gin_mlp(h, a0.reshape(n, width), a1.reshape(n, width),
                     W1f, b1f, W2, b2, concat=not edge_split)

    out2, pooled = _pool_mlp(h, batch, n_graphs, mlp_W1, mlp_b1,
                             mlp_W2, mlp_b2)
    return out2.reshape(-1), pooled


# speedup vs baseline: 1.2738x; 1.2738x over previous
import functools

import jax
import jax.numpy as jnp
from jax import lax
from jax.experimental import pallas as pl
from jax.experimental.pallas import tpu as pltpu
from jax.experimental.pallas import tpu_sc as plsc


_TILES = 16


def _seg_geometry(n_edges: int, edge_split: bool):
    n_workers = 2 * _TILES if edge_split else _TILES
    chunk = 112
    pnch = 18
    ep = n_edges // n_workers
    step = chunk * pnch
    ep_pad = -(-ep // step) * step
    phases = ep_pad // step
    return n_workers, chunk, ep_pad, phases, pnch


def _pad_idx(a, n_workers, ep_pad, n_mod, base):
    ep = a.shape[0] // n_workers
    a2 = a.reshape(n_workers, ep)
    pad = ep_pad - ep
    if pad == 0:
        return a2
    ii = lax.broadcasted_iota(jnp.int32, (n_workers, pad), 0)
    jj = lax.broadcasted_iota(jnp.int32, (n_workers, pad), 1)
    if base is None:
        padv = (ii * 37 + jj) % n_mod
    else:
        padv = base + ((ii + jj) % 16)
    return jnp.concatenate([a2, padv], axis=1)


@functools.lru_cache(maxsize=None)
def _make_seg_sum(n_nodes: int, n_edges: int, width: int, edge_split: bool):
    n_workers, chunk, ep_pad, phases, pnch = _seg_geometry(n_edges,
                                                           edge_split)
    nch = phases * pnch
    n_acc = n_nodes + 16
    rpt = n_nodes // _TILES
    assert rpt * _TILES == n_nodes
    vecs_per_row = chunk // 16

    mesh = plsc.VectorSubcoreMesh(core_axis_name="c", subcore_axis_name="s")

    @functools.partial(
        pl.kernel,
        out_type=[
            jax.ShapeDtypeStruct((_TILES, rpt, width), jnp.float32),
            jax.ShapeDtypeStruct((_TILES, rpt, width), jnp.float32),
        ],
        mesh=mesh,
        scratch_types=[
            pltpu.VMEM((pnch, chunk), jnp.int32),
            pltpu.VMEM((pnch, chunk), jnp.int32),
            pltpu.VMEM((chunk, width), jnp.float32),
            pltpu.VMEM((chunk, width), jnp.float32),
            pltpu.VMEM_SHARED((n_acc, width), jnp.float32),
            pltpu.SemaphoreType.DMA,
            pltpu.SemaphoreType.DMA,
        ],
    )
    def seg_kernel(hview, src_hbm, dst_hbm, zero_hbm, out_lo, out_hi,
                   src_v, dst_v, gb0, gb1, accum, sem0, sem1):
        c = lax.axis_index("c")
        s = lax.axis_index("s")
        w = c * _TILES + s if edge_split else s

        pltpu.sync_copy(zero_hbm, accum.at[pl.ds(s * rpt, rpt)])

        @pl.when(s == 0)
        def _():
            pltpu.sync_copy(zero_hbm.at[pl.ds(0, 16)],
                            accum.at[pl.ds(n_nodes, 16)])

        plsc.subcore_barrier()

        def _phase(p, _):
            pltpu.sync_copy(src_hbm.at[w, p], src_v)
            pltpu.sync_copy(dst_hbm.at[w, p], dst_v)

            if not edge_split:
                def _xform_row(j, _):
                    def _xform_vec(l, _):
                        v = src_v[j, pl.ds(l * 16, 16)]
                        src_v[j, pl.ds(l * 16, 16)] = v + v + c
                        return 0
                    return lax.fori_loop(0, vecs_per_row, _xform_vec, 0)
                lax.fori_loop(0, pnch, _xform_row, 0)

            pltpu.async_copy(hview.at[src_v.at[0]], gb0, sem0)
            pltpu.async_copy(hview.at[src_v.at[1]], gb1, sem1)

            def _body(kk, _):
                c0 = 2 * kk
                c1 = c0 + 1
                pltpu.make_async_copy(hview.at[src_v.at[c0]], gb0,
                                      sem0).wait()
                pltpu.sync_copy(gb0, accum.at[dst_v.at[c0]], add=True)
                nxt0 = jnp.minimum(c0 + 2, pnch - 1)
                pltpu.async_copy(hview.at[src_v.at[nxt0]], gb0, sem0)
                pltpu.make_async_copy(hview.at[src_v.at[c1]], gb1,
                                      sem1).wait()
                pltpu.sync_copy(gb1, accum.at[dst_v.at[c1]], add=True)
                nxt1 = jnp.minimum(c1 + 2, pnch - 1)
                pltpu.async_copy(hview.at[src_v.at[nxt1]], gb1, sem1)
                return 0

            lax.fori_loop(0, pnch // 2, _body, 0)
            pltpu.make_async_copy(hview.at[src_v.at[0]], gb0, sem0).wait()
            pltpu.make_async_copy(hview.at[src_v.at[0]], gb1, sem1).wait()
            return 0

        lax.fori_loop(0, phases, _phase, 0)
        plsc.subcore_barrier()

        @pl.when(c == 0)
        def _():
            pltpu.sync_copy(accum.at[pl.ds(s * rpt, rpt)], out_lo.at[s])

        @pl.when(c == 1)
        def _():
            pltpu.sync_copy(accum.at[pl.ds(s * rpt, rpt)], out_hi.at[s])

    return seg_kernel


_ROWS = 1000


def _gin_mlp_body(h_ref, a0_ref, a1_ref, w1_ref, b1_ref, w2_ref, b2_ref,
                  o_ref, *, concat):
    if concat:
        a = h_ref[...] + jnp.concatenate([a0_ref[...], a1_ref[...]], axis=1)
    else:
        a = h_ref[...] + a0_ref[...] + a1_ref[...]
    z = jnp.dot(a, w1_ref[...], preferred_element_type=jnp.float32)
    z = jnp.maximum(z + b1_ref[...], 0.0)
    o = jnp.dot(z, w2_ref[...], preferred_element_type=jnp.float32)
    o_ref[...] = jnp.maximum(o + b2_ref[...], 0.0)


def _gin_mlp(h, a0, a1, w1f, b1f, w2, b2, concat):
    n, d = h.shape
    hh = w2.shape[0]
    aw = a0.shape[1]
    grid = (n // _ROWS,)
    return pl.pallas_call(
        functools.partial(_gin_mlp_body, concat=concat),
        grid=grid,
        in_specs=[
            pl.BlockSpec((_ROWS, d), lambda i: (i, 0)),
            pl.BlockSpec((_ROWS, aw), lambda i: (i, 0)),
            pl.BlockSpec((_ROWS, aw), lambda i: (i, 0)),
            pl.BlockSpec((d, hh), lambda i: (0, 0)),
            pl.BlockSpec((1, hh), lambda i: (0, 0)),
            pl.BlockSpec((hh, hh), lambda i: (0, 0)),
            pl.BlockSpec((1, hh), lambda i: (0, 0)),
        ],
        out_specs=pl.BlockSpec((_ROWS, hh), lambda i: (i, 0)),
        out_shape=jax.ShapeDtypeStruct((n, hh), jnp.float32),
    )(h, a0, a1, w1f, b1f.reshape(1, -1), w2, b2.reshape(1, -1))


def _pool_mlp_body(h_ref, b_ref, w1_ref, b1_ref, w2_ref, b2_ref,
                   pooled_ref, out_ref):
    i = pl.program_id(0)
    nblk = pl.num_programs(0)
    g = pooled_ref.shape[0]
    seg = b_ref[0, 0, :]
    iota = lax.broadcasted_iota(jnp.int32, (g, seg.shape[0]), 0)
    onehot = (iota == seg[None, :]).astype(jnp.float32)
    part = jnp.dot(onehot, h_ref[...], preferred_element_type=jnp.float32)

    @pl.when(i == 0)
    def _():
        pooled_ref[...] = jnp.zeros_like(pooled_ref)

    pooled_ref[...] += part

    @pl.when(i == nblk - 1)
    def _():
        p = pooled_ref[...]
        z = jnp.dot(p, w1_ref[...], preferred_element_type=jnp.float32)
        z = jnp.maximum(z + b1_ref[...], 0.0)
        o = jnp.dot(z, w2_ref[...], preferred_element_type=jnp.float32)
        out_ref[...] = o + b2_ref[...]


def _pool_mlp(h, batch, n_graphs, w1, b1, w2, b2):
    n, d = h.shape
    out_d = w2.shape[1]
    grid = (n // _ROWS,)
    batch3 = batch.reshape(n // _ROWS, 1, _ROWS)
    pooled, out = pl.pallas_call(
        _pool_mlp_body,
        grid=grid,
        in_specs=[
            pl.BlockSpec((_ROWS, d), lambda i: (i, 0)),
            pl.BlockSpec((1, 1, _ROWS), lambda i: (i, 0, 0)),
            pl.BlockSpec((d, d), lambda i: (0, 0)),
            pl.BlockSpec((1, d), lambda i: (0, 0)),
            pl.BlockSpec((d, out_d), lambda i: (0, 0)),
            pl.BlockSpec((1, out_d), lambda i: (0, 0)),
        ],
        out_specs=[
            pl.BlockSpec((n_graphs, d), lambda i: (0, 0)),
            pl.BlockSpec((n_graphs, out_d), lambda i: (0, 0)),
        ],
        out_shape=[
            jax.ShapeDtypeStruct((n_graphs, d), jnp.float32),
            jax.ShapeDtypeStruct((n_graphs, out_d), jnp.float32),
        ],
    )(h, batch3, w1, b1.reshape(1, -1), w2, b2.reshape(1, -1))
    return out, pooled


def kernel(x, edge_index, batch,
           l0_W1, l0_b1, l0_g, l0_bb, l0_m, l0_v, l0_W2, l0_b2,
           l1_W1, l1_b1, l1_g, l1_bb, l1_m, l1_v, l1_W2, l1_b2,
           l2_W1, l2_b1, l2_g, l2_bb, l2_m, l2_v, l2_W2, l2_b2,
           mlp_W1, mlp_b1, mlp_W2, mlp_b2):
    n, _ = x.shape
    e = edge_index.shape[1]
    n_graphs = 64

    src, dst = edge_index[0], edge_index[1]

    layers = [
        (l0_W1, l0_b1, l0_g, l0_bb, l0_m, l0_v, l0_W2, l0_b2),
        (l1_W1, l1_b1, l1_g, l1_bb, l1_m, l1_v, l1_W2, l1_b2),
        (l2_W1, l2_b1, l2_g, l2_bb, l2_m, l2_v, l2_W2, l2_b2),
    ]

    h = x
    for (W1, b1, g, bb, m, v, W2, b2) in layers:
        d = h.shape[1]
        half = d // 2
        s = g / jnp.sqrt(v + 1e-5)
        W1f = W1 * s[None, :]
        b1f = (b1 - m) * s + bb
        edge_split = (half % 128 != 0)
        if edge_split:
            width = d
            hsrc = h
        else:
            width = half
            hsrc = h.reshape(2 * n, half)
        n_workers, chunk, ep_pad, phases, pnch = _seg_geometry(e, edge_split)
        src3d = _pad_idx(src, n_workers, ep_pad, n, None).reshape(
            n_workers, phases, pnch, chunk)
        dst3d = _pad_idx(dst, n_workers, ep_pad, n, n).reshape(
            n_workers, phases, pnch, chunk)
        zero = jnp.zeros((n // _TILES, width), jnp.float32)
        seg = _make_seg_sum(n, e, width, edge_split)
        a0, a1 = seg(hsrc, src3d, dst3d, zero)
        h = _gin_mlp(h, a0.reshape(n, width), a1.reshape(n, width),
                     W1f, b1f, W2, b2, concat=not edge_split)

    out2, pooled = _pool_mlp(h, batch, n_graphs, mlp_W1, mlp_b1,
                             mlp_W2, mlp_b2)
    return out2.reshape(-1), pooled


# speedup vs baseline: 1.3476x; 1.0579x over previous
import functools

import jax
import jax.numpy as jnp
from jax import lax
from jax.experimental import pallas as pl
from jax.experimental.pallas import tpu as pltpu
from jax.experimental.pallas import tpu_sc as plsc


_TILES = 16


def _seg_geometry(n_edges: int, edge_split: bool):
    n_workers = 2 * _TILES if edge_split else _TILES
    chunk = 112
    pnch = 30
    ep = n_edges // n_workers
    step = chunk * pnch
    ep_pad = -(-ep // step) * step
    phases = ep_pad // step
    return n_workers, chunk, ep_pad, phases, pnch


def _pad_idx(a, n_workers, ep_pad, n_mod, base):
    ep = a.shape[0] // n_workers
    a2 = a.reshape(n_workers, ep)
    pad = ep_pad - ep
    if pad == 0:
        return a2
    ii = lax.broadcasted_iota(jnp.int32, (n_workers, pad), 0)
    jj = lax.broadcasted_iota(jnp.int32, (n_workers, pad), 1)
    if base is None:
        padv = (ii * 37 + jj) % n_mod
    else:
        padv = base + ((ii + jj) % 16)
    return jnp.concatenate([a2, padv], axis=1)


@functools.lru_cache(maxsize=None)
def _make_seg_sum(n_nodes: int, n_edges: int, width: int, edge_split: bool):
    n_workers, chunk, ep_pad, phases, pnch = _seg_geometry(n_edges,
                                                           edge_split)
    nch = phases * pnch
    n_acc = n_nodes + 16
    rpt = n_nodes // _TILES
    assert rpt * _TILES == n_nodes

    mesh = plsc.VectorSubcoreMesh(core_axis_name="c", subcore_axis_name="s")

    @functools.partial(
        pl.kernel,
        out_type=[
            jax.ShapeDtypeStruct((_TILES, rpt, width), jnp.float32),
            jax.ShapeDtypeStruct((_TILES, rpt, width), jnp.float32),
        ],
        mesh=mesh,
        scratch_types=[
            pltpu.VMEM((pnch, chunk), jnp.int32),
            pltpu.VMEM((pnch, chunk), jnp.int32),
            pltpu.VMEM((chunk, width), jnp.float32),
            pltpu.VMEM((chunk, width), jnp.float32),
            pltpu.VMEM_SHARED((n_acc, width), jnp.float32),
            pltpu.SemaphoreType.DMA,
            pltpu.SemaphoreType.DMA,
        ],
    )
    def seg_kernel(hview, src_hbm, dst_hbm, zero_hbm, out_lo, out_hi,
                   src_v, dst_v, gb0, gb1, accum, sem0, sem1):
        c = lax.axis_index("c")
        s = lax.axis_index("s")
        w = c * _TILES + s if edge_split else s

        pltpu.sync_copy(zero_hbm, accum.at[pl.ds(s * rpt, rpt)])

        @pl.when(s == 0)
        def _():
            pltpu.sync_copy(zero_hbm.at[pl.ds(0, 16)],
                            accum.at[pl.ds(n_nodes, 16)])

        plsc.subcore_barrier()

        def _phase(p, _):
            if edge_split:
                pltpu.sync_copy(src_hbm.at[w, p], src_v)
            else:
                pltpu.sync_copy(src_hbm.at[c, s, p], src_v)
            pltpu.sync_copy(dst_hbm.at[w, p], dst_v)

            pltpu.async_copy(hview.at[src_v.at[0]], gb0, sem0)
            pltpu.async_copy(hview.at[src_v.at[1]], gb1, sem1)

            def _body(kk, _):
                c0 = 2 * kk
                c1 = c0 + 1
                pltpu.make_async_copy(hview.at[src_v.at[c0]], gb0,
                                      sem0).wait()
                pltpu.sync_copy(gb0, accum.at[dst_v.at[c0]], add=True)
                nxt0 = jnp.minimum(c0 + 2, pnch - 1)
                pltpu.async_copy(hview.at[src_v.at[nxt0]], gb0, sem0)
                pltpu.make_async_copy(hview.at[src_v.at[c1]], gb1,
                                      sem1).wait()
                pltpu.sync_copy(gb1, accum.at[dst_v.at[c1]], add=True)
                nxt1 = jnp.minimum(c1 + 2, pnch - 1)
                pltpu.async_copy(hview.at[src_v.at[nxt1]], gb1, sem1)
                return 0

            lax.fori_loop(0, pnch // 2, _body, 0)
            pltpu.make_async_copy(hview.at[src_v.at[0]], gb0, sem0).wait()
            pltpu.make_async_copy(hview.at[src_v.at[0]], gb1, sem1).wait()
            return 0

        lax.fori_loop(0, phases, _phase, 0)
        plsc.subcore_barrier()

        @pl.when(c == 0)
        def _():
            pltpu.sync_copy(accum.at[pl.ds(s * rpt, rpt)], out_lo.at[s])

        @pl.when(c == 1)
        def _():
            pltpu.sync_copy(accum.at[pl.ds(s * rpt, rpt)], out_hi.at[s])

    return seg_kernel


_ROWS = 1000


def _gin_mlp_body(h_ref, a0_ref, a1_ref, w1_ref, b1_ref, w2_ref, b2_ref,
                  o_ref, *, concat):
    if concat:
        a = h_ref[...] + jnp.concatenate([a0_ref[...], a1_ref[...]], axis=1)
    else:
        a = h_ref[...] + a0_ref[...] + a1_ref[...]
    z = jnp.dot(a, w1_ref[...], preferred_element_type=jnp.float32)
    z = jnp.maximum(z + b1_ref[...], 0.0)
    o = jnp.dot(z, w2_ref[...], preferred_element_type=jnp.float32)
    o_ref[...] = jnp.maximum(o + b2_ref[...], 0.0)


def _gin_mlp(h, a0, a1, w1f, b1f, w2, b2, concat):
    n, d = h.shape
    hh = w2.shape[0]
    aw = a0.shape[1]
    grid = (n // _ROWS,)
    return pl.pallas_call(
        functools.partial(_gin_mlp_body, concat=concat),
        grid=grid,
        in_specs=[
            pl.BlockSpec((_ROWS, d), lambda i: (i, 0)),
            pl.BlockSpec((_ROWS, aw), lambda i: (i, 0)),
            pl.BlockSpec((_ROWS, aw), lambda i: (i, 0)),
            pl.BlockSpec((d, hh), lambda i: (0, 0)),
            pl.BlockSpec((1, hh), lambda i: (0, 0)),
            pl.BlockSpec((hh, hh), lambda i: (0, 0)),
            pl.BlockSpec((1, hh), lambda i: (0, 0)),
        ],
        out_specs=pl.BlockSpec((_ROWS, hh), lambda i: (i, 0)),
        out_shape=jax.ShapeDtypeStruct((n, hh), jnp.float32),
    )(h, a0, a1, w1f, b1f.reshape(1, -1), w2, b2.reshape(1, -1))


def _pool_mlp_body(h_ref, b_ref, w1_ref, b1_ref, w2_ref, b2_ref,
                   pooled_ref, out_ref):
    i = pl.program_id(0)
    nblk = pl.num_programs(0)
    g = pooled_ref.shape[0]
    seg = b_ref[0, 0, :]
    iota = lax.broadcasted_iota(jnp.int32, (g, seg.shape[0]), 0)
    onehot = (iota == seg[None, :]).astype(jnp.float32)
    part = jnp.dot(onehot, h_ref[...], preferred_element_type=jnp.float32)

    @pl.when(i == 0)
    def _():
        pooled_ref[...] = jnp.zeros_like(pooled_ref)

    pooled_ref[...] += part

    @pl.when(i == nblk - 1)
    def _():
        p = pooled_ref[...]
        z = jnp.dot(p, w1_ref[...], preferred_element_type=jnp.float32)
        z = jnp.maximum(z + b1_ref[...], 0.0)
        o = jnp.dot(z, w2_ref[...], preferred_element_type=jnp.float32)
        out_ref[...] = o + b2_ref[...]


def _pool_mlp(h, batch, n_graphs, w1, b1, w2, b2):
    n, d = h.shape
    out_d = w2.shape[1]
    grid = (n // _ROWS,)
    batch3 = batch.reshape(n // _ROWS, 1, _ROWS)
    pooled, out = pl.pallas_call(
        _pool_mlp_body,
        grid=grid,
        in_specs=[
            pl.BlockSpec((_ROWS, d), lambda i: (i, 0)),
            pl.BlockSpec((1, 1, _ROWS), lambda i: (i, 0, 0)),
            pl.BlockSpec((d, d), lambda i: (0, 0)),
            pl.BlockSpec((1, d), lambda i: (0, 0)),
            pl.BlockSpec((d, out_d), lambda i: (0, 0)),
            pl.BlockSpec((1, out_d), lambda i: (0, 0)),
        ],
        out_specs=[
            pl.BlockSpec((n_graphs, d), lambda i: (0, 0)),
            pl.BlockSpec((n_graphs, out_d), lambda i: (0, 0)),
        ],
        out_shape=[
            jax.ShapeDtypeStruct((n_graphs, d), jnp.float32),
            jax.ShapeDtypeStruct((n_graphs, out_d), jnp.float32),
        ],
    )(h, batch3, w1, b1.reshape(1, -1), w2, b2.reshape(1, -1))
    return out, pooled


def kernel(x, edge_index, batch,
           l0_W1, l0_b1, l0_g, l0_bb, l0_m, l0_v, l0_W2, l0_b2,
           l1_W1, l1_b1, l1_g, l1_bb, l1_m, l1_v, l1_W2, l1_b2,
           l2_W1, l2_b1, l2_g, l2_bb, l2_m, l2_v, l2_W2, l2_b2,
           mlp_W1, mlp_b1, mlp_W2, mlp_b2):
    n, _ = x.shape
    e = edge_index.shape[1]
    n_graphs = 64

    src, dst = edge_index[0], edge_index[1]

    layers = [
        (l0_W1, l0_b1, l0_g, l0_bb, l0_m, l0_v, l0_W2, l0_b2),
        (l1_W1, l1_b1, l1_g, l1_bb, l1_m, l1_v, l1_W2, l1_b2),
        (l2_W1, l2_b1, l2_g, l2_bb, l2_m, l2_v, l2_W2, l2_b2),
    ]

    h = x
    for (W1, b1, g, bb, m, v, W2, b2) in layers:
        d = h.shape[1]
        half = d // 2
        s = g / jnp.sqrt(v + 1e-5)
        W1f = W1 * s[None, :]
        b1f = (b1 - m) * s + bb
        edge_split = (half % 128 != 0)
        if edge_split:
            width = d
            hsrc = h
        else:
            width = half
            hsrc = h.reshape(2 * n, half)
        n_workers, chunk, ep_pad, phases, pnch = _seg_geometry(e, edge_split)
        srcp = _pad_idx(src, n_workers, ep_pad, n, None)
        if edge_split:
            src3d = srcp.reshape(n_workers, phases, pnch, chunk)
        else:
            src3d = jnp.stack([srcp * 2, srcp * 2 + 1]).reshape(
                2, n_workers, phases, pnch, chunk)
        dst3d = _pad_idx(dst, n_workers, ep_pad, n, n).reshape(
            n_workers, phases, pnch, chunk)
        zero = jnp.zeros((n // _TILES, width), jnp.float32)
        seg = _make_seg_sum(n, e, width, edge_split)
        a0, a1 = seg(hsrc, src3d, dst3d, zero)
        h = _gin_mlp(h, a0.reshape(n, width), a1.reshape(n, width),
                     W1f, b1f, W2, b2, concat=not edge_split)

    out2, pooled = _pool_mlp(h, batch, n_graphs, mlp_W1, mlp_b1,
                             mlp_W2, mlp_b2)
    return out2.reshape(-1), pooled


# speedup vs baseline: 1.3678x; 1.0150x over previous
import functools

import jax
import jax.numpy as jnp
from jax import lax
from jax.experimental import pallas as pl
from jax.experimental.pallas import tpu as pltpu
from jax.experimental.pallas import tpu_sc as plsc


_TILES = 16


def _seg_geometry(n_edges: int, edge_split: bool):
    n_workers = 2 * _TILES if edge_split else _TILES
    chunk = 112
    pnch = 18
    ep = n_edges // n_workers
    step = chunk * pnch
    ep_pad = -(-ep // step) * step
    phases = ep_pad // step
    return n_workers, chunk, ep_pad, phases, pnch


def _pad_idx(a, n_workers, ep_pad, n_mod, base):
    ep = a.shape[0] // n_workers
    a2 = a.reshape(n_workers, ep)
    pad = ep_pad - ep
    if pad == 0:
        return a2
    ii = lax.broadcasted_iota(jnp.int32, (n_workers, pad), 0)
    jj = lax.broadcasted_iota(jnp.int32, (n_workers, pad), 1)
    if base is None:
        padv = (ii * 37 + jj) % n_mod
    else:
        padv = base + ((ii + jj) % 16)
    return jnp.concatenate([a2, padv], axis=1)


@functools.lru_cache(maxsize=None)
def _make_seg_sum(n_nodes: int, n_edges: int, width: int, edge_split: bool):
    n_workers, chunk, ep_pad, phases, pnch = _seg_geometry(n_edges,
                                                           edge_split)
    nch = phases * pnch
    n_acc = -(-n_nodes // 128) * 128
    rpt = n_acc // _TILES
    assert rpt % 8 == 0 and n_acc >= n_nodes + 16

    mesh = plsc.VectorSubcoreMesh(core_axis_name="c", subcore_axis_name="s")

    @functools.partial(
        pl.kernel,
        out_type=[
            jax.ShapeDtypeStruct((n_acc, width), jnp.float32),
            jax.ShapeDtypeStruct((n_acc, width), jnp.float32),
        ],
        mesh=mesh,
        scratch_types=[
            pltpu.VMEM((pnch, chunk), jnp.int32),
            pltpu.VMEM((pnch, chunk), jnp.int32),
            pltpu.VMEM((chunk, width), jnp.float32),
            pltpu.VMEM((chunk, width), jnp.float32),
            pltpu.VMEM_SHARED((n_acc, width), jnp.float32),
            pltpu.SemaphoreType.DMA,
            pltpu.SemaphoreType.DMA,
        ],
    )
    def seg_kernel(hview, src_hbm, dst_hbm, zero_hbm, out_lo, out_hi,
                   src_v, dst_v, gb0, gb1, accum, sem0, sem1):
        c = lax.axis_index("c")
        s = lax.axis_index("s")
        w = c * _TILES + s if edge_split else s

        pltpu.sync_copy(zero_hbm, accum.at[pl.ds(s * rpt, rpt)])
        plsc.subcore_barrier()

        def _phase(p, _):
            if edge_split:
                pltpu.sync_copy(src_hbm.at[w, p], src_v)
            else:
                pltpu.sync_copy(src_hbm.at[c, s, p], src_v)
            pltpu.sync_copy(dst_hbm.at[w, p], dst_v)

            pltpu.async_copy(hview.at[src_v.at[0]], gb0, sem0)
            pltpu.async_copy(hview.at[src_v.at[1]], gb1, sem1)

            def _body(kk, _):
                c0 = 2 * kk
                c1 = c0 + 1
                pltpu.make_async_copy(hview.at[src_v.at[c0]], gb0,
                                      sem0).wait()
                pltpu.sync_copy(gb0, accum.at[dst_v.at[c0]], add=True)
                nxt0 = jnp.minimum(c0 + 2, pnch - 1)
                pltpu.async_copy(hview.at[src_v.at[nxt0]], gb0, sem0)
                pltpu.make_async_copy(hview.at[src_v.at[c1]], gb1,
                                      sem1).wait()
                pltpu.sync_copy(gb1, accum.at[dst_v.at[c1]], add=True)
                nxt1 = jnp.minimum(c1 + 2, pnch - 1)
                pltpu.async_copy(hview.at[src_v.at[nxt1]], gb1, sem1)
                return 0

            lax.fori_loop(0, pnch // 2, _body, 0)
            pltpu.make_async_copy(hview.at[src_v.at[0]], gb0, sem0).wait()
            pltpu.make_async_copy(hview.at[src_v.at[0]], gb1, sem1).wait()
            return 0

        lax.fori_loop(0, phases, _phase, 0)
        plsc.subcore_barrier()

        @pl.when(c == 0)
        def _():
            pltpu.sync_copy(accum.at[pl.ds(s * rpt, rpt)],
                            out_lo.at[pl.ds(s * rpt, rpt)])

        @pl.when(c == 1)
        def _():
            pltpu.sync_copy(accum.at[pl.ds(s * rpt, rpt)],
                            out_hi.at[pl.ds(s * rpt, rpt)])

    return seg_kernel


_ROWS = 1000


def _gin_mlp_body(h_ref, a0_ref, a1_ref, w1_ref, b1_ref, w2_ref, b2_ref,
                  o_ref, *, concat, d, hh):
    h = h_ref[...].reshape(_ROWS, d)
    if concat:
        a = h + jnp.concatenate([a0_ref[...], a1_ref[...]], axis=1)
    else:
        a = h + a0_ref[...] + a1_ref[...]
    z = jnp.dot(a, w1_ref[...], preferred_element_type=jnp.float32)
    z = jnp.maximum(z + b1_ref[...], 0.0)
    o = jnp.dot(z, w2_ref[...], preferred_element_type=jnp.float32)
    o = jnp.maximum(o + b2_ref[...], 0.0)
    o_ref[...] = o.reshape(o_ref.shape)


def _gin_mlp(h, a0, a1, w1f, b1f, w2, b2, concat):
    d = w1f.shape[0]
    hh = w2.shape[0]
    n = h.shape[0] * h.shape[1] // d
    aw = a0.shape[1]
    rh = _ROWS * d // 128
    ro = _ROWS * hh // 128
    grid = (n // _ROWS,)
    return pl.pallas_call(
        functools.partial(_gin_mlp_body, concat=concat, d=d, hh=hh),
        grid=grid,
        in_specs=[
            pl.BlockSpec((rh, 128), lambda i: (i, 0)),
            pl.BlockSpec((_ROWS, aw), lambda i: (i, 0)),
            pl.BlockSpec((_ROWS, aw), lambda i: (i, 0)),
            pl.BlockSpec((d, hh), lambda i: (0, 0)),
            pl.BlockSpec((1, hh), lambda i: (0, 0)),
            pl.BlockSpec((hh, hh), lambda i: (0, 0)),
            pl.BlockSpec((1, hh), lambda i: (0, 0)),
        ],
        out_specs=pl.BlockSpec((ro, 128), lambda i: (i, 0)),
        out_shape=jax.ShapeDtypeStruct((n * hh // 128, 128), jnp.float32),
    )(h, a0, a1, w1f, b1f.reshape(1, -1), w2, b2.reshape(1, -1))


def _pool_mlp_body(h_ref, b_ref, w1_ref, b1_ref, w2_ref, b2_ref,
                   pooled_ref, out_ref, *, d):
    i = pl.program_id(0)
    nblk = pl.num_programs(0)
    g = pooled_ref.shape[0]
    seg = b_ref[0, 0, :]
    h = h_ref[...].reshape(seg.shape[0], d)
    iota = lax.broadcasted_iota(jnp.int32, (g, seg.shape[0]), 0)
    onehot = (iota == seg[None, :]).astype(jnp.float32)
    part = jnp.dot(onehot, h, preferred_element_type=jnp.float32)

    @pl.when(i == 0)
    def _():
        pooled_ref[...] = jnp.zeros_like(pooled_ref)

    pooled_ref[...] += part

    @pl.when(i == nblk - 1)
    def _():
        p = pooled_ref[...]
        z = jnp.dot(p, w1_ref[...], preferred_element_type=jnp.float32)
        z = jnp.maximum(z + b1_ref[...], 0.0)
        o = jnp.dot(z, w2_ref[...], preferred_element_type=jnp.float32)
        out_ref[...] = o + b2_ref[...]


def _pool_mlp(h, batch, n_graphs, w1, b1, w2, b2):
    d = w1.shape[0]
    n = h.shape[0] * h.shape[1] // d
    out_d = w2.shape[1]
    rh = _ROWS * d // 128
    grid = (n // _ROWS,)
    batch3 = batch.reshape(n // _ROWS, 1, _ROWS)
    pooled, out = pl.pallas_call(
        functools.partial(_pool_mlp_body, d=d),
        grid=grid,
        in_specs=[
            pl.BlockSpec((rh, 128), lambda i: (i, 0)),
            pl.BlockSpec((1, 1, _ROWS), lambda i: (i, 0, 0)),
            pl.BlockSpec((d, d), lambda i: (0, 0)),
            pl.BlockSpec((1, d), lambda i: (0, 0)),
            pl.BlockSpec((d, out_d), lambda i: (0, 0)),
            pl.BlockSpec((1, out_d), lambda i: (0, 0)),
        ],
        out_specs=[
            pl.BlockSpec((n_graphs, d), lambda i: (0, 0)),
            pl.BlockSpec((n_graphs, out_d), lambda i: (0, 0)),
        ],
        out_shape=[
            jax.ShapeDtypeStruct((n_graphs, d), jnp.float32),
            jax.ShapeDtypeStruct((n_graphs, out_d), jnp.float32),
        ],
    )(h, batch3, w1, b1.reshape(1, -1), w2, b2.reshape(1, -1))
    return out, pooled


def kernel(x, edge_index, batch,
           l0_W1, l0_b1, l0_g, l0_bb, l0_m, l0_v, l0_W2, l0_b2,
           l1_W1, l1_b1, l1_g, l1_bb, l1_m, l1_v, l1_W2, l1_b2,
           l2_W1, l2_b1, l2_g, l2_bb, l2_m, l2_v, l2_W2, l2_b2,
           mlp_W1, mlp_b1, mlp_W2, mlp_b2):
    n, _ = x.shape
    e = edge_index.shape[1]
    n_graphs = 64

    src, dst = edge_index[0], edge_index[1]

    layers = [
        (l0_W1, l0_b1, l0_g, l0_bb, l0_m, l0_v, l0_W2, l0_b2),
        (l1_W1, l1_b1, l1_g, l1_bb, l1_m, l1_v, l1_W2, l1_b2),
        (l2_W1, l2_b1, l2_g, l2_bb, l2_m, l2_v, l2_W2, l2_b2),
    ]

    h = x
    for (W1, b1, g, bb, m, v, W2, b2) in layers:
        d = W1.shape[0]
        half = d // 2
        s = g / jnp.sqrt(v + 1e-5)
        W1f = W1 * s[None, :]
        b1f = (b1 - m) * s + bb
        edge_split = (half % 128 != 0)
        width = d if edge_split else half
        n_workers, chunk, ep_pad, phases, pnch = _seg_geometry(e, edge_split)
        srcp = _pad_idx(src, n_workers, ep_pad, n, None)
        if edge_split:
            src3d = srcp.reshape(n_workers, phases, pnch, chunk)
        else:
            src3d = jnp.stack([srcp * 2, srcp * 2 + 1]).reshape(
                2, n_workers, phases, pnch, chunk)
        dst3d = _pad_idx(dst, n_workers, ep_pad, n, n).reshape(
            n_workers, phases, pnch, chunk)
        n_acc = -(-n // 128) * 128
        zero = jnp.zeros((n_acc // _TILES, width), jnp.float32)
        seg = _make_seg_sum(n, e, width, edge_split)
        a0, a1 = seg(h, src3d, dst3d, zero)
        h = _gin_mlp(h, a0, a1, W1f, b1f, W2, b2, concat=not edge_split)

    out2, pooled = _pool_mlp(h, batch, n_graphs, mlp_W1, mlp_b1,
                             mlp_W2, mlp_b2)
    return out2.reshape(-1), pooled


# speedup vs baseline: 1.4410x; 1.0535x over previous
import functools

import jax
import jax.numpy as jnp
from jax import lax
from jax.experimental import pallas as pl
from jax.experimental.pallas import tpu as pltpu
from jax.experimental.pallas import tpu_sc as plsc


_TILES = 16


def _seg_geometry(n_edges: int, edge_split: bool):
    n_workers = 2 * _TILES if edge_split else _TILES
    chunk = 112
    pnch = 30
    ep = n_edges // n_workers
    step = chunk * pnch
    ep_pad = -(-ep // step) * step
    phases = ep_pad // step
    return n_workers, chunk, ep_pad, phases, pnch


def _pad_idx(a, n_workers, ep_pad, n_mod, base):
    ep = a.shape[0] // n_workers
    a2 = a.reshape(n_workers, ep)
    pad = ep_pad - ep
    if pad == 0:
        return a2
    ii = lax.broadcasted_iota(jnp.int32, (n_workers, pad), 0)
    jj = lax.broadcasted_iota(jnp.int32, (n_workers, pad), 1)
    if base is None:
        padv = (ii * 37 + jj) % n_mod
    else:
        padv = base + ((ii + jj) % 16)
    return jnp.concatenate([a2, padv], axis=1)


@functools.lru_cache(maxsize=None)
def _make_seg_sum(n_nodes: int, n_edges: int, width: int, edge_split: bool):
    n_workers, chunk, ep_pad, phases, pnch = _seg_geometry(n_edges,
                                                           edge_split)
    nch = phases * pnch
    n_acc = -(-n_nodes // 128) * 128
    rpt = n_acc // _TILES
    assert rpt % 8 == 0 and n_acc >= n_nodes + 16

    mesh = plsc.VectorSubcoreMesh(core_axis_name="c", subcore_axis_name="s")

    @functools.partial(
        pl.kernel,
        out_type=[
            jax.ShapeDtypeStruct((n_acc, width), jnp.float32),
            jax.ShapeDtypeStruct((n_acc, width), jnp.float32),
        ],
        mesh=mesh,
        scratch_types=[
            pltpu.VMEM((pnch, chunk), jnp.int32),
            pltpu.VMEM((pnch, chunk), jnp.int32),
            pltpu.VMEM((chunk, width), jnp.float32),
            pltpu.VMEM((chunk, width), jnp.float32),
            pltpu.VMEM_SHARED((n_acc, width), jnp.float32),
            pltpu.SemaphoreType.DMA,
            pltpu.SemaphoreType.DMA,
        ],
    )
    def seg_kernel(hview, src_hbm, dst_hbm, zero_hbm, out_lo, out_hi,
                   src_v, dst_v, gb0, gb1, accum, sem0, sem1):
        c = lax.axis_index("c")
        s = lax.axis_index("s")
        w = c * _TILES + s if edge_split else s

        pltpu.sync_copy(zero_hbm, accum.at[pl.ds(s * rpt, rpt)])
        plsc.subcore_barrier()

        def _phase(p, _):
            if edge_split:
                pltpu.sync_copy(src_hbm.at[w, p], src_v)
            else:
                pltpu.sync_copy(src_hbm.at[c, s, p], src_v)
            pltpu.sync_copy(dst_hbm.at[w, p], dst_v)

            pltpu.async_copy(hview.at[src_v.at[0]], gb0, sem0)
            pltpu.async_copy(hview.at[src_v.at[1]], gb1, sem1)

            def _body(kk, _):
                c0 = 2 * kk
                c1 = c0 + 1
                pltpu.make_async_copy(hview.at[src_v.at[c0]], gb0,
                                      sem0).wait()
                pltpu.sync_copy(gb0, accum.at[dst_v.at[c0]], add=True)
                nxt0 = jnp.minimum(c0 + 2, pnch - 1)
                pltpu.async_copy(hview.at[src_v.at[nxt0]], gb0, sem0)
                pltpu.make_async_copy(hview.at[src_v.at[c1]], gb1,
                                      sem1).wait()
                pltpu.sync_copy(gb1, accum.at[dst_v.at[c1]], add=True)
                nxt1 = jnp.minimum(c1 + 2, pnch - 1)
                pltpu.async_copy(hview.at[src_v.at[nxt1]], gb1, sem1)
                return 0

            lax.fori_loop(0, pnch // 2, _body, 0)
            pltpu.make_async_copy(hview.at[src_v.at[0]], gb0, sem0).wait()
            pltpu.make_async_copy(hview.at[src_v.at[0]], gb1, sem1).wait()
            return 0

        lax.fori_loop(0, phases, _phase, 0)
        plsc.subcore_barrier()

        @pl.when(c == 0)
        def _():
            pltpu.sync_copy(accum.at[pl.ds(s * rpt, rpt)],
                            out_lo.at[pl.ds(s * rpt, rpt)])

        @pl.when(c == 1)
        def _():
            pltpu.sync_copy(accum.at[pl.ds(s * rpt, rpt)],
                            out_hi.at[pl.ds(s * rpt, rpt)])

    return seg_kernel


_ROWS = 1000


def _gin_mlp_body(h_ref, a0_ref, a1_ref, w1_ref, b1_ref, w2_ref, b2_ref,
                  o_ref, *, concat, d, hh):
    h = h_ref[...].reshape(_ROWS, d)
    if concat:
        a = h + jnp.concatenate([a0_ref[...], a1_ref[...]], axis=1)
    else:
        a = h + a0_ref[...] + a1_ref[...]
    z = jnp.dot(a, w1_ref[...], preferred_element_type=jnp.float32)
    z = jnp.maximum(z + b1_ref[...], 0.0)
    o = jnp.dot(z, w2_ref[...], preferred_element_type=jnp.float32)
    o = jnp.maximum(o + b2_ref[...], 0.0)
    o_ref[...] = o.reshape(o_ref.shape)


def _gin_mlp(h, a0, a1, w1f, b1f, w2, b2, concat):
    d = w1f.shape[0]
    hh = w2.shape[0]
    n = h.shape[0] * h.shape[1] // d
    aw = a0.shape[1]
    rh = _ROWS * d // 128
    ro = _ROWS * hh // 128
    grid = (n // _ROWS,)
    return pl.pallas_call(
        functools.partial(_gin_mlp_body, concat=concat, d=d, hh=hh),
        grid=grid,
        in_specs=[
            pl.BlockSpec((rh, 128), lambda i: (i, 0)),
            pl.BlockSpec((_ROWS, aw), lambda i: (i, 0)),
            pl.BlockSpec((_ROWS, aw), lambda i: (i, 0)),
            pl.BlockSpec((d, hh), lambda i: (0, 0)),
            pl.BlockSpec((1, hh), lambda i: (0, 0)),
            pl.BlockSpec((hh, hh), lambda i: (0, 0)),
            pl.BlockSpec((1, hh), lambda i: (0, 0)),
        ],
        out_specs=pl.BlockSpec((ro, 128), lambda i: (i, 0)),
        out_shape=jax.ShapeDtypeStruct((n * hh // 128, 128), jnp.float32),
    )(h, a0, a1, w1f, b1f.reshape(1, -1), w2, b2.reshape(1, -1))


def _pool_mlp_body(h_ref, b_ref, w1_ref, b1_ref, w2_ref, b2_ref,
                   pooled_ref, out_ref, *, d):
    i = pl.program_id(0)
    nblk = pl.num_programs(0)
    g = pooled_ref.shape[0]
    seg = b_ref[0, 0, :]
    h = h_ref[...].reshape(seg.shape[0], d)
    iota = lax.broadcasted_iota(jnp.int32, (g, seg.shape[0]), 0)
    onehot = (iota == seg[None, :]).astype(jnp.float32)
    part = jnp.dot(onehot, h, preferred_element_type=jnp.float32)

    @pl.when(i == 0)
    def _():
        pooled_ref[...] = jnp.zeros_like(pooled_ref)

    pooled_ref[...] += part

    @pl.when(i == nblk - 1)
    def _():
        p = pooled_ref[...]
        z = jnp.dot(p, w1_ref[...], preferred_element_type=jnp.float32)
        z = jnp.maximum(z + b1_ref[...], 0.0)
        o = jnp.dot(z, w2_ref[...], preferred_element_type=jnp.float32)
        out_ref[...] = o + b2_ref[...]


def _pool_mlp(h, batch, n_graphs, w1, b1, w2, b2):
    d = w1.shape[0]
    n = h.shape[0] * h.shape[1] // d
    out_d = w2.shape[1]
    rh = _ROWS * d // 128
    grid = (n // _ROWS,)
    batch3 = batch.reshape(n // _ROWS, 1, _ROWS)
    pooled, out = pl.pallas_call(
        functools.partial(_pool_mlp_body, d=d),
        grid=grid,
        in_specs=[
            pl.BlockSpec((rh, 128), lambda i: (i, 0)),
            pl.BlockSpec((1, 1, _ROWS), lambda i: (i, 0, 0)),
            pl.BlockSpec((d, d), lambda i: (0, 0)),
            pl.BlockSpec((1, d), lambda i: (0, 0)),
            pl.BlockSpec((d, out_d), lambda i: (0, 0)),
            pl.BlockSpec((1, out_d), lambda i: (0, 0)),
        ],
        out_specs=[
            pl.BlockSpec((n_graphs, d), lambda i: (0, 0)),
            pl.BlockSpec((n_graphs, out_d), lambda i: (0, 0)),
        ],
        out_shape=[
            jax.ShapeDtypeStruct((n_graphs, d), jnp.float32),
            jax.ShapeDtypeStruct((n_graphs, out_d), jnp.float32),
        ],
    )(h, batch3, w1, b1.reshape(1, -1), w2, b2.reshape(1, -1))
    return out, pooled


def kernel(x, edge_index, batch,
           l0_W1, l0_b1, l0_g, l0_bb, l0_m, l0_v, l0_W2, l0_b2,
           l1_W1, l1_b1, l1_g, l1_bb, l1_m, l1_v, l1_W2, l1_b2,
           l2_W1, l2_b1, l2_g, l2_bb, l2_m, l2_v, l2_W2, l2_b2,
           mlp_W1, mlp_b1, mlp_W2, mlp_b2):
    n, _ = x.shape
    e = edge_index.shape[1]
    n_graphs = 64

    src, dst = edge_index[0], edge_index[1]

    layers = [
        (l0_W1, l0_b1, l0_g, l0_bb, l0_m, l0_v, l0_W2, l0_b2),
        (l1_W1, l1_b1, l1_g, l1_bb, l1_m, l1_v, l1_W2, l1_b2),
        (l2_W1, l2_b1, l2_g, l2_bb, l2_m, l2_v, l2_W2, l2_b2),
    ]

    h = x
    for (W1, b1, g, bb, m, v, W2, b2) in layers:
        d = W1.shape[0]
        half = d // 2
        s = g / jnp.sqrt(v + 1e-5)
        W1f = W1 * s[None, :]
        b1f = (b1 - m) * s + bb
        edge_split = (half % 128 != 0)
        width = d if edge_split else half
        n_workers, chunk, ep_pad, phases, pnch = _seg_geometry(e, edge_split)
        srcp = _pad_idx(src, n_workers, ep_pad, n, None)
        if edge_split:
            src3d = srcp.reshape(n_workers, phases, pnch, chunk)
        else:
            src3d = jnp.stack([srcp * 2, srcp * 2 + 1]).reshape(
                2, n_workers, phases, pnch, chunk)
        dst3d = _pad_idx(dst, n_workers, ep_pad, n, n).reshape(
            n_workers, phases, pnch, chunk)
        n_acc = -(-n // 128) * 128
        zero = jnp.zeros((n_acc // _TILES, width), jnp.float32)
        seg = _make_seg_sum(n, e, width, edge_split)
        a0, a1 = seg(h, src3d, dst3d, zero)
        h = _gin_mlp(h, a0, a1, W1f, b1f, W2, b2, concat=not edge_split)

    out2, pooled = _pool_mlp(h, batch, n_graphs, mlp_W1, mlp_b1,
                             mlp_W2, mlp_b2)
    return out2.reshape(-1), pooled


# speedup vs baseline: 1.4540x; 1.0090x over previous
import functools

import jax
import jax.numpy as jnp
from jax import lax
from jax.experimental import pallas as pl
from jax.experimental.pallas import tpu as pltpu
from jax.experimental.pallas import tpu_sc as plsc


_TILES = 16


def _seg_geometry(n_edges: int, edge_split: bool):
    n_workers = 2 * _TILES if edge_split else _TILES
    chunk = 120
    pnch = 28
    ep = n_edges // n_workers
    step = chunk * pnch
    ep_pad = -(-ep // step) * step
    phases = ep_pad // step
    return n_workers, chunk, ep_pad, phases, pnch


def _pad_idx(a, n_workers, ep_pad, n_mod, base):
    ep = a.shape[0] // n_workers
    a2 = a.reshape(n_workers, ep)
    pad = ep_pad - ep
    if pad == 0:
        return a2
    ii = lax.broadcasted_iota(jnp.int32, (n_workers, pad), 0)
    jj = lax.broadcasted_iota(jnp.int32, (n_workers, pad), 1)
    if base is None:
        padv = (ii * 37 + jj) % n_mod
    else:
        padv = base + ((ii + jj) % 16)
    return jnp.concatenate([a2, padv], axis=1)


@functools.lru_cache(maxsize=None)
def _make_seg_sum(n_nodes: int, n_edges: int, width: int, edge_split: bool):
    n_workers, chunk, ep_pad, phases, pnch = _seg_geometry(n_edges,
                                                           edge_split)
    nch = phases * pnch
    n_acc = -(-n_nodes // 128) * 128
    rpt = n_acc // _TILES
    assert rpt % 8 == 0 and n_acc >= n_nodes + 16

    mesh = plsc.VectorSubcoreMesh(core_axis_name="c", subcore_axis_name="s")

    @functools.partial(
        pl.kernel,
        out_type=[
            jax.ShapeDtypeStruct((n_acc, width), jnp.float32),
            jax.ShapeDtypeStruct((n_acc, width), jnp.float32),
        ],
        mesh=mesh,
        scratch_types=[
            pltpu.VMEM((pnch, chunk), jnp.int32),
            pltpu.VMEM((pnch, chunk), jnp.int32),
            pltpu.VMEM((chunk, width), jnp.float32),
            pltpu.VMEM((chunk, width), jnp.float32),
            pltpu.VMEM_SHARED((n_acc, width), jnp.float32),
            pltpu.SemaphoreType.DMA,
            pltpu.SemaphoreType.DMA,
        ],
    )
    def seg_kernel(hview, src_hbm, dst_hbm, zero_hbm, out_lo, out_hi,
                   src_v, dst_v, gb0, gb1, accum, sem0, sem1):
        c = lax.axis_index("c")
        s = lax.axis_index("s")
        w = c * _TILES + s if edge_split else s

        pltpu.sync_copy(zero_hbm, accum.at[pl.ds(s * rpt, rpt)])
        plsc.subcore_barrier()

        def _phase(p, _):
            if edge_split:
                pltpu.sync_copy(src_hbm.at[w, p], src_v)
            else:
                pltpu.sync_copy(src_hbm.at[c, s, p], src_v)
            pltpu.sync_copy(dst_hbm.at[w, p], dst_v)

            pltpu.async_copy(hview.at[src_v.at[0]], gb0, sem0)
            pltpu.async_copy(hview.at[src_v.at[1]], gb1, sem1)

            def _body(kk, _):
                c0 = 2 * kk
                c1 = c0 + 1
                pltpu.make_async_copy(hview.at[src_v.at[c0]], gb0,
                                      sem0).wait()
                pltpu.sync_copy(gb0, accum.at[dst_v.at[c0]], add=True)
                nxt0 = jnp.minimum(c0 + 2, pnch - 1)
                pltpu.async_copy(hview.at[src_v.at[nxt0]], gb0, sem0)
                pltpu.make_async_copy(hview.at[src_v.at[c1]], gb1,
                                      sem1).wait()
                pltpu.sync_copy(gb1, accum.at[dst_v.at[c1]], add=True)
                nxt1 = jnp.minimum(c1 + 2, pnch - 1)
                pltpu.async_copy(hview.at[src_v.at[nxt1]], gb1, sem1)
                return 0

            lax.fori_loop(0, pnch // 2, _body, 0)
            pltpu.make_async_copy(hview.at[src_v.at[0]], gb0, sem0).wait()
            pltpu.make_async_copy(hview.at[src_v.at[0]], gb1, sem1).wait()
            return 0

        lax.fori_loop(0, phases, _phase, 0)
        plsc.subcore_barrier()

        @pl.when(c == 0)
        def _():
            pltpu.sync_copy(accum.at[pl.ds(s * rpt, rpt)],
                            out_lo.at[pl.ds(s * rpt, rpt)])

        @pl.when(c == 1)
        def _():
            pltpu.sync_copy(accum.at[pl.ds(s * rpt, rpt)],
                            out_hi.at[pl.ds(s * rpt, rpt)])

    return seg_kernel


_ROWS = 1000


def _gin_mlp_body(h_ref, a0_ref, a1_ref, w1_ref, b1_ref, w2_ref, b2_ref,
                  o_ref, *, concat, d, hh):
    h = h_ref[...].reshape(_ROWS, d)
    if concat:
        a = h + jnp.concatenate([a0_ref[...], a1_ref[...]], axis=1)
    else:
        a = h + a0_ref[...] + a1_ref[...]
    z = jnp.dot(a, w1_ref[...], preferred_element_type=jnp.float32)
    z = jnp.maximum(z + b1_ref[...], 0.0)
    o = jnp.dot(z, w2_ref[...], preferred_element_type=jnp.float32)
    o = jnp.maximum(o + b2_ref[...], 0.0)
    o_ref[...] = o.reshape(o_ref.shape)


def _gin_mlp(h, a0, a1, w1f, b1f, w2, b2, concat):
    d = w1f.shape[0]
    hh = w2.shape[0]
    n = h.shape[0] * h.shape[1] // d
    aw = a0.shape[1]
    rh = _ROWS * d // 128
    ro = _ROWS * hh // 128
    grid = (n // _ROWS,)
    return pl.pallas_call(
        functools.partial(_gin_mlp_body, concat=concat, d=d, hh=hh),
        grid=grid,
        in_specs=[
            pl.BlockSpec((rh, 128), lambda i: (i, 0)),
            pl.BlockSpec((_ROWS, aw), lambda i: (i, 0)),
            pl.BlockSpec((_ROWS, aw), lambda i: (i, 0)),
            pl.BlockSpec((d, hh), lambda i: (0, 0)),
            pl.BlockSpec((1, hh), lambda i: (0, 0)),
            pl.BlockSpec((hh, hh), lambda i: (0, 0)),
            pl.BlockSpec((1, hh), lambda i: (0, 0)),
        ],
        out_specs=pl.BlockSpec((ro, 128), lambda i: (i, 0)),
        out_shape=jax.ShapeDtypeStruct((n * hh // 128, 128), jnp.float32),
    )(h, a0, a1, w1f, b1f.reshape(1, -1), w2, b2.reshape(1, -1))


def _pool_mlp_body(h_ref, b_ref, w1_ref, b1_ref, w2_ref, b2_ref,
                   pooled_ref, out_ref, *, d):
    i = pl.program_id(0)
    nblk = pl.num_programs(0)
    g = pooled_ref.shape[0]
    seg = b_ref[0, 0, :]
    h = h_ref[...].reshape(seg.shape[0], d)
    iota = lax.broadcasted_iota(jnp.int32, (g, seg.shape[0]), 0)
    onehot = (iota == seg[None, :]).astype(jnp.float32)
    part = jnp.dot(onehot, h, preferred_element_type=jnp.float32)

    @pl.when(i == 0)
    def _():
        pooled_ref[...] = jnp.zeros_like(pooled_ref)

    pooled_ref[...] += part

    @pl.when(i == nblk - 1)
    def _():
        p = pooled_ref[...]
        z = jnp.dot(p, w1_ref[...], preferred_element_type=jnp.float32)
        z = jnp.maximum(z + b1_ref[...], 0.0)
        o = jnp.dot(z, w2_ref[...], preferred_element_type=jnp.float32)
        out_ref[...] = o + b2_ref[...]


def _pool_mlp(h, batch, n_graphs, w1, b1, w2, b2):
    d = w1.shape[0]
    n = h.shape[0] * h.shape[1] // d
    out_d = w2.shape[1]
    rh = _ROWS * d // 128
    grid = (n // _ROWS,)
    batch3 = batch.reshape(n // _ROWS, 1, _ROWS)
    pooled, out = pl.pallas_call(
        functools.partial(_pool_mlp_body, d=d),
        grid=grid,
        in_specs=[
            pl.BlockSpec((rh, 128), lambda i: (i, 0)),
            pl.BlockSpec((1, 1, _ROWS), lambda i: (i, 0, 0)),
            pl.BlockSpec((d, d), lambda i: (0, 0)),
            pl.BlockSpec((1, d), lambda i: (0, 0)),
            pl.BlockSpec((d, out_d), lambda i: (0, 0)),
            pl.BlockSpec((1, out_d), lambda i: (0, 0)),
        ],
        out_specs=[
            pl.BlockSpec((n_graphs, d), lambda i: (0, 0)),
            pl.BlockSpec((n_graphs, out_d), lambda i: (0, 0)),
        ],
        out_shape=[
            jax.ShapeDtypeStruct((n_graphs, d), jnp.float32),
            jax.ShapeDtypeStruct((n_graphs, out_d), jnp.float32),
        ],
    )(h, batch3, w1, b1.reshape(1, -1), w2, b2.reshape(1, -1))
    return out, pooled


def kernel(x, edge_index, batch,
           l0_W1, l0_b1, l0_g, l0_bb, l0_m, l0_v, l0_W2, l0_b2,
           l1_W1, l1_b1, l1_g, l1_bb, l1_m, l1_v, l1_W2, l1_b2,
           l2_W1, l2_b1, l2_g, l2_bb, l2_m, l2_v, l2_W2, l2_b2,
           mlp_W1, mlp_b1, mlp_W2, mlp_b2):
    n, _ = x.shape
    e = edge_index.shape[1]
    n_graphs = 64

    src, dst = edge_index[0], edge_index[1]

    layers = [
        (l0_W1, l0_b1, l0_g, l0_bb, l0_m, l0_v, l0_W2, l0_b2),
        (l1_W1, l1_b1, l1_g, l1_bb, l1_m, l1_v, l1_W2, l1_b2),
        (l2_W1, l2_b1, l2_g, l2_bb, l2_m, l2_v, l2_W2, l2_b2),
    ]

    h = x
    for (W1, b1, g, bb, m, v, W2, b2) in layers:
        d = W1.shape[0]
        half = d // 2
        s = g / jnp.sqrt(v + 1e-5)
        W1f = W1 * s[None, :]
        b1f = (b1 - m) * s + bb
        edge_split = (half % 128 != 0)
        width = d if edge_split else half
        n_workers, chunk, ep_pad, phases, pnch = _seg_geometry(e, edge_split)
        srcp = _pad_idx(src, n_workers, ep_pad, n, None)
        if edge_split:
            src3d = srcp.reshape(n_workers, phases, pnch, chunk)
        else:
            src3d = jnp.stack([srcp * 2, srcp * 2 + 1]).reshape(
                2, n_workers, phases, pnch, chunk)
        dst3d = _pad_idx(dst, n_workers, ep_pad, n, n).reshape(
            n_workers, phases, pnch, chunk)
        n_acc = -(-n // 128) * 128
        zero = jnp.zeros((n_acc // _TILES, width), jnp.float32)
        seg = _make_seg_sum(n, e, width, edge_split)
        a0, a1 = seg(h, src3d, dst3d, zero)
        h = _gin_mlp(h, a0, a1, W1f, b1f, W2, b2, concat=not edge_split)

    out2, pooled = _pool_mlp(h, batch, n_graphs, mlp_W1, mlp_b1,
                             mlp_W2, mlp_b2)
    return out2.reshape(-1), pooled
